# Initial kernel scaffold; baseline (speedup 1.0000x reference)
import jax
import jax.numpy as jnp
from jax.experimental import pallas as pl


def kernel(x, gate_w):
    raise NotImplementedError("write your pallas kernel here")


# speedup vs baseline: 3.4571x; 3.4571x over previous
import functools

import jax
import jax.numpy as jnp
from jax.experimental import pallas as pl
from jax.experimental.pallas import tpu as pltpu

_C_IN = 2048
_NUM_EXPERTS = 16
_N_TOKENS = 16384
_BLOCK = 1024


def _switch_gate_body(x_ref, w_ref, out_ref, loss_ref, acc_ref):
    i = pl.program_id(0)

    x = x_ref[...]
    w = w_ref[...]
    logits = jax.lax.dot_general(
        x, w, (((1,), (1,)), ((), ())), preferred_element_type=jnp.float32
    )

    m = jnp.max(logits, axis=-1, keepdims=True)
    e = jnp.exp(logits - m)
    probs = e / jnp.sum(e, axis=-1, keepdims=True)

    amax = jnp.argmax(logits, axis=-1)
    eids = jax.lax.broadcasted_iota(jnp.int32, logits.shape, 1)
    mask = (eids == amax[:, None]).astype(jnp.float32)
    masked = probs * mask
    out_ref[...] = masked

    @pl.when(i == 0)
    def _init():
        acc_ref[...] = jnp.zeros_like(acc_ref)

    acc_ref[0, :] += jnp.sum(masked, axis=0)
    acc_ref[1, :] += jnp.sum(mask, axis=0)

    @pl.when(i == pl.num_programs(0) - 1)
    def _finish():
        s = acc_ref[0, :]
        c = acc_ref[1, :]
        n = jnp.float32(_N_TOKENS)
        loss_ref[...] = jnp.sum(s * c)[None, None] * (_NUM_EXPERTS / (n * n))


@functools.partial(jax.jit, static_argnames=("interpret",))
def kernel(x, gate_w, interpret=False):
    n_tokens, c_in = x.shape
    num_experts = gate_w.shape[0]
    grid = (n_tokens // _BLOCK,)
    masked, loss = pl.pallas_call(
        _switch_gate_body,
        grid=grid,
        in_specs=[
            pl.BlockSpec((_BLOCK, c_in), lambda i: (i, 0)),
            pl.BlockSpec((num_experts, c_in), lambda i: (0, 0)),
        ],
        out_specs=[
            pl.BlockSpec((_BLOCK, num_experts), lambda i: (i, 0)),
            pl.BlockSpec((1, 1), lambda i: (0, 0)),
        ],
        out_shape=[
            jax.ShapeDtypeStruct((n_tokens, num_experts), jnp.float32),
            jax.ShapeDtypeStruct((1, 1), jnp.float32),
        ],
        scratch_shapes=[pltpu.VMEM((2, num_experts), jnp.float32)],
        interpret=interpret,
    )(x, gate_w)
    return masked, loss[0, 0]


# speedup vs baseline: 3.5068x; 1.0144x over previous
import functools

import jax
import jax.numpy as jnp
from jax.experimental import pallas as pl
from jax.experimental.pallas import tpu as pltpu

_C_IN = 2048
_NUM_EXPERTS = 16
_N_TOKENS = 16384
_BLOCK = 2048


def _switch_gate_body(x_ref, w_ref, out_ref, loss_ref, acc_ref):
    i = pl.program_id(0)

    x = x_ref[...]
    w = w_ref[...]
    logits = jax.lax.dot_general(
        x, w, (((1,), (1,)), ((), ())), preferred_element_type=jnp.float32
    )

    m = jnp.max(logits, axis=-1, keepdims=True)
    e = jnp.exp(logits - m)
    probs = e / jnp.sum(e, axis=-1, keepdims=True)

    amax = jnp.argmax(logits, axis=-1)
    eids = jax.lax.broadcasted_iota(jnp.int32, logits.shape, 1)
    mask = (eids == amax[:, None]).astype(jnp.float32)
    masked = probs * mask
    out_ref[...] = masked

    @pl.when(i == 0)
    def _init():
        acc_ref[...] = jnp.zeros_like(acc_ref)

    acc_ref[0, :] += jnp.sum(masked, axis=0)
    acc_ref[1, :] += jnp.sum(mask, axis=0)

    @pl.when(i == pl.num_programs(0) - 1)
    def _finish():
        s = acc_ref[0, :]
        c = acc_ref[1, :]
        n = jnp.float32(_N_TOKENS)
        loss_ref[...] = jnp.sum(s * c)[None, None] * (_NUM_EXPERTS / (n * n))


@functools.partial(jax.jit, static_argnames=("interpret",))
def kernel(x, gate_w, interpret=False):
    n_tokens, c_in = x.shape
    num_experts = gate_w.shape[0]
    grid = (n_tokens // _BLOCK,)
    masked, loss = pl.pallas_call(
        _switch_gate_body,
        grid=grid,
        in_specs=[
            pl.BlockSpec((_BLOCK, c_in), lambda i: (i, 0)),
            pl.BlockSpec((num_experts, c_in), lambda i: (0, 0)),
        ],
        out_specs=[
            pl.BlockSpec((_BLOCK, num_experts), lambda i: (i, 0)),
            pl.BlockSpec((1, 1), lambda i: (0, 0)),
        ],
        out_shape=[
            jax.ShapeDtypeStruct((n_tokens, num_experts), jnp.float32),
            jax.ShapeDtypeStruct((1, 1), jnp.float32),
        ],
        scratch_shapes=[pltpu.VMEM((2, num_experts), jnp.float32)],
        interpret=interpret,
    )(x, gate_w)
    return masked, loss[0, 0]
